# Initial kernel scaffold; baseline (speedup 1.0000x reference)
import jax
import jax.numpy as jnp
from jax.experimental import pallas as pl


def kernel(x):
    raise NotImplementedError("write your pallas kernel here")


# speedup vs baseline: 8.7675x; 8.7675x over previous
import jax
import jax.numpy as jnp
from jax.experimental import pallas as pl


def _body(x_ref, o_ref):
    c = pl.program_id(0)

    @pl.when(c == 0)
    def _():
        o_ref[...] = jnp.zeros_like(o_ref)

    col = jax.lax.broadcasted_iota(jnp.int32, o_ref.shape, 1)
    o_ref[...] = jnp.where(col == c, x_ref[:, 0:1], o_ref[...])


def kernel(x):
    return pl.pallas_call(
        _body,
        grid=(32,),
        in_specs=[pl.BlockSpec((1024, 128), lambda c: (0, 16 * c))],
        out_specs=pl.BlockSpec((1024, 32), lambda c: (0, 0)),
        out_shape=jax.ShapeDtypeStruct((1024, 32), jnp.float32),
    )(x)
